# baseline (speedup 1.0000x reference)
import functools

import jax
import jax.numpy as jnp
from jax import lax
from jax.experimental import pallas as pl
from jax.experimental.pallas import tpu as pltpu
from jax.experimental.pallas import tpu_sc as plsc

_CHUNK = 64
_NBUF = 8
_C0_CHUNKS_PER_SUB = 8


@functools.cache
def _build(B, C, K, P_mu, P_al):
    info = plsc.get_sparse_core_info()
    NC, NS = info.num_cores, info.num_subcores
    NW = NC * NS
    b_per_w = B // NW
    assert B % (NW * _CHUNK) == 0
    n_chunks = b_per_w // _CHUNK
    cc0 = _C0_CHUNKS_PER_SUB
    cc1 = 2 * n_chunks - cc0
    assert 0 < cc1
    n_planes = 2 * P_mu + P_al
    R_mu = P_mu * C
    R_al = P_al * C

    mesh = plsc.VectorSubcoreMesh(core_axis_name="c", subcore_axis_name="s")

    @functools.partial(
        pl.kernel,
        mesh=mesh,
        out_type=[
            jax.ShapeDtypeStruct((P_mu, B, K), jnp.float32),
            jax.ShapeDtypeStruct((P_mu, B, K), jnp.float32),
            jax.ShapeDtypeStruct((P_al, B, K), jnp.float32),
        ],
        scratch_types=[
            pltpu.VMEM((max(cc0, cc1), n_planes, _CHUNK), jnp.int32),
            pltpu.VMEM((_NBUF, _CHUNK, K), jnp.float32),
            pltpu.VMEM_SHARED((2 * R_mu + R_al, K), jnp.float32),
            pltpu.SemaphoreType.DMA((_NBUF,)),
            pltpu.SemaphoreType.DMA((_NBUF,)),
        ],
    )
    def gather_kernel(ids_hbm, mu_hbm, sg_hbm, al_hbm,
                      mu_out, sg_out, al_out,
                      idx_v, buf_v, spm, sem_g, sem_w):
        cid = lax.axis_index("c")
        sid = lax.axis_index("s")
        wid = sid * NC + cid

        half = NS // 2

        def _stage(src, rows, dst_base, lane):
            per = ((rows + half - 1) // half + 7) // 8 * 8
            rem = rows - per * (half - 1)
            assert rem > 0 and rem % 8 == 0

            @pl.when((lane >= 0) & (lane < half - 1))
            def _bulk():
                pltpu.sync_copy(
                    src.at[pl.ds(lane * per, per)],
                    spm.at[pl.ds(dst_base + lane * per, per)])

            @pl.when(lane == half - 1)
            def _tail():
                off = per * (half - 1)
                pltpu.sync_copy(
                    src.at[pl.ds(off, rem)],
                    spm.at[pl.ds(dst_base + off, rem)])

        _stage(mu_hbm, R_mu, 0, sid)
        _stage(sg_hbm, R_mu, R_mu, sid - half)
        _stage(al_hbm, R_al, 2 * R_mu, sid)

        plsc.subcore_barrier()

        outs = ([(mu_out, p) for p in range(P_mu)]
                + [(sg_out, p) for p in range(P_mu)]
                + [(al_out, p) for p in range(P_al)])

        def run_tasks(my_chunks, chunk_base):
            pltpu.sync_copy(ids_hbm.at[pl.ds(chunk_base, my_chunks)],
                            idx_v.at[pl.ds(0, my_chunks)])
            tasks = [(pp, j) for j in range(my_chunks)
                     for pp in range(n_planes)]

            def start_gather(t):
                pp, j = tasks[t]
                b = t % _NBUF
                return pltpu.async_copy(
                    spm.at[idx_v.at[j, pp]], buf_v.at[b], sem_g.at[b])

            def start_write(t):
                pp, j = tasks[t]
                out, p = outs[pp]
                b = t % _NBUF
                base = (chunk_base + j) * _CHUNK
                return pltpu.async_copy(
                    buf_v.at[b], out.at[p, pl.ds(base, _CHUNK)], sem_w.at[b])

            n_tasks = len(tasks)
            gather_cps = [None] * _NBUF
            write_cps = [None] * _NBUF
            for t in range(min(_NBUF - 1, n_tasks)):
                gather_cps[t % _NBUF] = start_gather(t)
            for t in range(n_tasks):
                b = t % _NBUF
                gather_cps[b].wait()
                write_cps[b] = start_write(t)
                nxt = t + _NBUF - 1
                if nxt < n_tasks:
                    nb = nxt % _NBUF
                    if write_cps[nb] is not None:
                        write_cps[nb].wait()
                        write_cps[nb] = None
                    gather_cps[nb] = start_gather(nxt)
            for b in range(_NBUF):
                if write_cps[b] is not None:
                    write_cps[b].wait()

        c0, c1 = cc0, cc1
        assert c0 * NS + c1 * NS == B // _CHUNK

        @pl.when(cid == 0)
        def _run_core0():
            run_tasks(c0, sid * c0)

        @pl.when(cid == 1)
        def _run_core1():
            run_tasks(c1, c0 * NS + sid * c1)

    return gather_kernel


def kernel(category_ids, mu, sigma, alpha):
    B = category_ids.shape[0]
    C, K, P_mu = mu.shape
    P_al = alpha.shape[2]
    n_planes = 2 * P_mu + P_al
    ids = category_ids.astype(jnp.int32)
    offs = jnp.arange(n_planes, dtype=jnp.int32) * C
    ids7 = ids[:, None] + offs[None, :]
    ids7 = jnp.transpose(ids7.reshape(B // _CHUNK, _CHUNK, n_planes),
                         (0, 2, 1))
    mu_p = jnp.transpose(mu, (2, 0, 1)).reshape(P_mu * C, K)
    sg_p = jnp.transpose(sigma, (2, 0, 1)).reshape(P_mu * C, K)
    al_p = jnp.transpose(alpha, (2, 0, 1)).reshape(P_al * C, K)
    f = _build(B, C, K, P_mu, P_al)
    mu_o, sg_o, al_o = f(ids7, mu_p, sg_p, al_p)
    return (jnp.transpose(mu_o, (1, 2, 0)),
            jnp.transpose(sg_o, (1, 2, 0)),
            jnp.transpose(al_o, (1, 2, 0)))


# speedup vs baseline: 1.0356x; 1.0356x over previous
import functools

import jax
import jax.numpy as jnp
from jax import lax
from jax.experimental import pallas as pl
from jax.experimental.pallas import tpu as pltpu
from jax.experimental.pallas import tpu_sc as plsc

_CHUNK = 128
_NBUF = 4
_C0_CHUNKS_PER_SUB = 4


@functools.cache
def _build(B, C, K, P_mu, P_al):
    info = plsc.get_sparse_core_info()
    NC, NS = info.num_cores, info.num_subcores
    NW = NC * NS
    b_per_w = B // NW
    assert B % (NW * _CHUNK) == 0
    n_chunks = b_per_w // _CHUNK
    cc0 = _C0_CHUNKS_PER_SUB
    cc1 = 2 * n_chunks - cc0
    assert 0 < cc1
    n_planes = 2 * P_mu + P_al
    R_mu = P_mu * C
    R_al = P_al * C

    mesh = plsc.VectorSubcoreMesh(core_axis_name="c", subcore_axis_name="s")

    @functools.partial(
        pl.kernel,
        mesh=mesh,
        out_type=[
            jax.ShapeDtypeStruct((P_mu, B, K), jnp.float32),
            jax.ShapeDtypeStruct((P_mu, B, K), jnp.float32),
            jax.ShapeDtypeStruct((P_al, B, K), jnp.float32),
        ],
        scratch_types=[
            pltpu.VMEM((max(cc0, cc1), n_planes, _CHUNK), jnp.int32),
            pltpu.VMEM((_NBUF, _CHUNK, K), jnp.float32),
            pltpu.VMEM_SHARED((2 * R_mu + R_al, K), jnp.float32),
            pltpu.SemaphoreType.DMA((_NBUF,)),
            pltpu.SemaphoreType.DMA((_NBUF,)),
        ],
    )
    def gather_kernel(ids_hbm, mu_hbm, sg_hbm, al_hbm,
                      mu_out, sg_out, al_out,
                      idx_v, buf_v, spm, sem_g, sem_w):
        cid = lax.axis_index("c")
        sid = lax.axis_index("s")
        wid = sid * NC + cid

        half = NS // 2

        def _stage(src, rows, dst_base, lane):
            per = ((rows + half - 1) // half + 7) // 8 * 8
            rem = rows - per * (half - 1)
            assert rem > 0 and rem % 8 == 0

            @pl.when((lane >= 0) & (lane < half - 1))
            def _bulk():
                pltpu.sync_copy(
                    src.at[pl.ds(lane * per, per)],
                    spm.at[pl.ds(dst_base + lane * per, per)])

            @pl.when(lane == half - 1)
            def _tail():
                off = per * (half - 1)
                pltpu.sync_copy(
                    src.at[pl.ds(off, rem)],
                    spm.at[pl.ds(dst_base + off, rem)])

        _stage(mu_hbm, R_mu, 0, sid)
        _stage(sg_hbm, R_mu, R_mu, sid - half)
        _stage(al_hbm, R_al, 2 * R_mu, sid)

        plsc.subcore_barrier()

        outs = ([(mu_out, p) for p in range(P_mu)]
                + [(sg_out, p) for p in range(P_mu)]
                + [(al_out, p) for p in range(P_al)])

        def run_tasks(my_chunks, chunk_base):
            pltpu.sync_copy(ids_hbm.at[pl.ds(chunk_base, my_chunks)],
                            idx_v.at[pl.ds(0, my_chunks)])
            tasks = [(pp, j) for j in range(my_chunks)
                     for pp in range(n_planes)]

            def start_gather(t):
                pp, j = tasks[t]
                b = t % _NBUF
                return pltpu.async_copy(
                    spm.at[idx_v.at[j, pp]], buf_v.at[b], sem_g.at[b])

            def start_write(t):
                pp, j = tasks[t]
                out, p = outs[pp]
                b = t % _NBUF
                base = (chunk_base + j) * _CHUNK
                return pltpu.async_copy(
                    buf_v.at[b], out.at[p, pl.ds(base, _CHUNK)], sem_w.at[b])

            n_tasks = len(tasks)
            gather_cps = [None] * _NBUF
            write_cps = [None] * _NBUF
            for t in range(min(_NBUF - 1, n_tasks)):
                gather_cps[t % _NBUF] = start_gather(t)
            for t in range(n_tasks):
                b = t % _NBUF
                gather_cps[b].wait()
                write_cps[b] = start_write(t)
                nxt = t + _NBUF - 1
                if nxt < n_tasks:
                    nb = nxt % _NBUF
                    if write_cps[nb] is not None:
                        write_cps[nb].wait()
                        write_cps[nb] = None
                    gather_cps[nb] = start_gather(nxt)
            for b in range(_NBUF):
                if write_cps[b] is not None:
                    write_cps[b].wait()

        c0, c1 = cc0, cc1
        assert c0 * NS + c1 * NS == B // _CHUNK

        @pl.when(cid == 0)
        def _run_core0():
            run_tasks(c0, sid * c0)

        @pl.when(cid == 1)
        def _run_core1():
            run_tasks(c1, c0 * NS + sid * c1)

    return gather_kernel


def kernel(category_ids, mu, sigma, alpha):
    B = category_ids.shape[0]
    C, K, P_mu = mu.shape
    P_al = alpha.shape[2]
    n_planes = 2 * P_mu + P_al
    ids = category_ids.astype(jnp.int32)
    offs = jnp.arange(n_planes, dtype=jnp.int32) * C
    ids7 = ids[:, None] + offs[None, :]
    ids7 = jnp.transpose(ids7.reshape(B // _CHUNK, _CHUNK, n_planes),
                         (0, 2, 1))
    mu_p = jnp.transpose(mu, (2, 0, 1)).reshape(P_mu * C, K)
    sg_p = jnp.transpose(sigma, (2, 0, 1)).reshape(P_mu * C, K)
    al_p = jnp.transpose(alpha, (2, 0, 1)).reshape(P_al * C, K)
    f = _build(B, C, K, P_mu, P_al)
    mu_o, sg_o, al_o = f(ids7, mu_p, sg_p, al_p)
    return (jnp.transpose(mu_o, (1, 2, 0)),
            jnp.transpose(sg_o, (1, 2, 0)),
            jnp.transpose(al_o, (1, 2, 0)))
